# Initial kernel scaffold; baseline (speedup 1.0000x reference)
import jax
import jax.numpy as jnp
from jax.experimental import pallas as pl


def kernel(x_constraint, x_operator, edge_index_co, edge_index_oc, batch_constraint, batch_operator, params):
    raise NotImplementedError("write your pallas kernel here")


# speedup vs baseline: 2.4826x; 2.4826x over previous
import functools

import jax
import jax.numpy as jnp
from jax import lax
from jax.experimental import pallas as pl
from jax.experimental.pallas import tpu as pltpu
from jax.experimental.pallas import tpu_sc as plsc

N = 10000
NP = 10240
H = 128
E = 320000
NSUB = 16
NSTEP = 160
NCH = 32
EPAD = NSUB * NSTEP * 128
NG = 64
BLK = 1280
NBLK = NP // BLK
F32 = jnp.float32
_P = jax.lax.Precision.HIGHEST


def _dot(a, b):
    return jnp.dot(a, b, preferred_element_type=F32, precision=_P)


def _seg_sum_sc(msg_all, src_idx, dst_idx):
    mesh = plsc.VectorSubcoreMesh(core_axis_name="c", subcore_axis_name="s")

    @functools.partial(
        pl.kernel,
        out_type=jax.ShapeDtypeStruct((2, NP, H), F32),
        mesh=mesh,
        scratch_types=[
            pltpu.VMEM((NCH, 128), jnp.int32),
            pltpu.VMEM((NCH, 128), jnp.int32),
            pltpu.VMEM((128, H), F32),
            pltpu.VMEM_SHARED((NP, H), F32),
            pltpu.SemaphoreType.DMA,
        ],
    )
    def k(msg_hbm, src_hbm, dst_hbm, acc_out,
          src_v, dst_v, rows_v, acc_sp, sem):
        c = lax.axis_index("c")
        s = lax.axis_index("s")
        t_src = 1 - c
        z16 = jnp.zeros((16,), F32)

        def zero_rows(i, carry):
            rows_v[i // 8, pl.ds((i % 8) * 16, 16)] = z16
            return carry

        lax.fori_loop(0, 128 * 8, zero_rows, 0)

        for b in range(5):
            pltpu.sync_copy(rows_v, acc_sp.at[pl.ds(s * 640 + b * 128, 128)])
        plsc.subcore_barrier()

        def chunk(jj, carry):
            pltpu.sync_copy(src_hbm.at[c, s, pl.ds(jj * NCH, NCH)], src_v)
            pltpu.sync_copy(dst_hbm.at[c, s, pl.ds(jj * NCH, NCH)], dst_v)

            def step(j, carry2):
                pltpu.async_copy(msg_hbm.at[t_src].at[src_v.at[j]], rows_v,
                                 sem).wait()
                pltpu.sync_copy(rows_v, acc_sp.at[dst_v.at[j]], add=True)
                return carry2

            lax.fori_loop(0, NCH, step, 0)
            return carry

        lax.fori_loop(0, NSTEP // NCH, chunk, 0)
        plsc.subcore_barrier()

        for b in range(5):
            pltpu.sync_copy(acc_sp.at[pl.ds(s * 640 + b * 128, 128)],
                            acc_out.at[c, pl.ds(s * 640 + b * 128, 128)])

    return k(msg_all, src_idx, dst_idx)


def _tc_count(dst_flat):
    cblk = 2048

    def body(d_r, o_r):
        i = pl.program_id(1)
        d = d_r[0]
        g = lax.broadcasted_iota(jnp.int32, (1, 128), 1)
        ohr = (lax.shift_right_logical(d, 7) == g).astype(jnp.bfloat16)
        ohc = (lax.bitwise_and(d, 127) == g).astype(jnp.bfloat16)
        dn = (((0,), (0,)), ((), ()))
        cc = lax.dot_general(ohr, ohc, dn, preferred_element_type=F32)

        @pl.when(i == 0)
        def _():
            o_r[0] = cc

        @pl.when(i > 0)
        def _():
            o_r[0] += cc

    return pl.pallas_call(
        body,
        grid=(2, EPAD // cblk),
        in_specs=[pl.BlockSpec((1, cblk, 1), lambda t, i: (t, i, 0))],
        out_specs=pl.BlockSpec((1, 128, 128), lambda t, i: (t, 0, 0)),
        out_shape=jax.ShapeDtypeStruct((2, 128, 128), F32),
    )(dst_flat)


def _tc_init(x_all, lin_W, lin_b, W_msg0):
    def body(x_r, w_r, b_r, wm_r, x1_r, msg_r):
        y = _dot(x_r[0], w_r[0]) + b_r[0]
        x1_r[0] = y
        msg_r[0] = _dot(y, wm_r[0])

    return pl.pallas_call(
        body,
        grid=(2, NBLK),
        in_specs=[
            pl.BlockSpec((1, BLK, H), lambda t, i: (t, i, 0)),
            pl.BlockSpec((1, H, H), lambda t, i: (t, 0, 0)),
            pl.BlockSpec((1, 1, H), lambda t, i: (t, 0, 0)),
            pl.BlockSpec((1, H, H), lambda t, i: (t, 0, 0)),
        ],
        out_specs=[
            pl.BlockSpec((1, BLK, H), lambda t, i: (t, i, 0)),
            pl.BlockSpec((1, BLK, H), lambda t, i: (t, i, 0)),
        ],
        out_shape=[
            jax.ShapeDtypeStruct((2, NP, H), F32),
            jax.ShapeDtypeStruct((2, NP, H), F32),
        ],
    )(x_all, lin_W, lin_b, W_msg0)


def _tc_conv_gru(x_all, acc, cntr, W_self, b_conv, Wi, bi, bh, W_msg_next):
    with_msg = W_msg_next is not None

    def body(x_r, a_r, c_r, ws_r, bc_r, wi_r, bi_r, bh_r, *rest):
        if with_msg:
            wm_r, xo_r, mo_r = rest
        else:
            (xo_r,) = rest
        m = a_r[0] * (1.0 / jnp.maximum(c_r[0], 1.0))
        h = jax.nn.relu(_dot(x_r[0], ws_r[0]) + m + bc_r[0])
        gi = _dot(h, wi_r[0]) + bi_r[0]
        bhv = bh_r[0]
        r = jax.nn.sigmoid(gi[:, :H] + bhv[:, :H])
        z = jax.nn.sigmoid(gi[:, H:2 * H] + bhv[:, H:2 * H])
        n = jnp.tanh(gi[:, 2 * H:] + r * bhv[:, 2 * H:])
        xn = (1.0 - z) * n
        xo_r[0] = xn
        if with_msg:
            mo_r[0] = _dot(xn, wm_r[0])

    in_specs = [
        pl.BlockSpec((1, BLK, H), lambda t, i: (t, i, 0)),
        pl.BlockSpec((1, BLK, H), lambda t, i: (t, i, 0)),
        pl.BlockSpec((1, BLK, 1), lambda t, i: (t, i, 0)),
        pl.BlockSpec((1, H, H), lambda t, i: (t, 0, 0)),
        pl.BlockSpec((1, 1, H), lambda t, i: (t, 0, 0)),
        pl.BlockSpec((1, H, 3 * H), lambda t, i: (t, 0, 0)),
        pl.BlockSpec((1, 1, 3 * H), lambda t, i: (t, 0, 0)),
        pl.BlockSpec((1, 1, 3 * H), lambda t, i: (t, 0, 0)),
    ]
    out_specs = [pl.BlockSpec((1, BLK, H), lambda t, i: (t, i, 0))]
    out_shape = [jax.ShapeDtypeStruct((2, NP, H), F32)]
    args = [x_all, acc, cntr, W_self, b_conv, Wi, bi, bh]
    if with_msg:
        in_specs.append(pl.BlockSpec((1, H, H), lambda t, i: (t, 0, 0)))
        out_specs.append(pl.BlockSpec((1, BLK, H), lambda t, i: (t, i, 0)))
        out_shape.append(jax.ShapeDtypeStruct((2, NP, H), F32))
        args.append(W_msg_next)

    res = pl.pallas_call(
        body, grid=(2, NBLK), in_specs=in_specs,
        out_specs=out_specs, out_shape=out_shape,
    )(*args)
    return res if with_msg else (res[0], None)


def _tc_pool(x_all, bids):
    def body(x_r, b_r, ps_r, pc_r):
        i = pl.program_id(1)
        g = lax.broadcasted_iota(jnp.int32, (1, 128), 1)
        oh = (b_r[0] == g).astype(F32)
        dn = (((0,), (0,)), ((), ()))
        ps = lax.dot_general(oh, x_r[0], dn,
                             preferred_element_type=F32, precision=_P)
        pc = lax.dot_general(oh, jnp.ones((BLK, 128), F32), dn,
                             preferred_element_type=F32, precision=_P)

        @pl.when(i == 0)
        def _():
            ps_r[0] = ps
            pc_r[0] = pc

        @pl.when(i > 0)
        def _():
            ps_r[0] += ps
            pc_r[0] += pc

    return pl.pallas_call(
        body,
        grid=(2, NBLK),
        in_specs=[
            pl.BlockSpec((1, BLK, H), lambda t, i: (t, i, 0)),
            pl.BlockSpec((1, BLK, 1), lambda t, i: (t, i, 0)),
        ],
        out_specs=[
            pl.BlockSpec((1, 128, 128), lambda t, i: (t, 0, 0)),
            pl.BlockSpec((1, 128, 128), lambda t, i: (t, 0, 0)),
        ],
        out_shape=[
            jax.ShapeDtypeStruct((2, 128, 128), F32),
            jax.ShapeDtypeStruct((2, 128, 128), F32),
        ],
    )(x_all, bids)


def _tc_mlp(psum, pcnt, lin_W, lin_b, out_Wp, out_bp):
    def body(ps_r, pc_r, w_r, b_r, ow_r, ob_r, o_r):
        mc = ps_r[0, :NG, :] / jnp.maximum(pc_r[0, :NG, :], 1.0)
        mo = ps_r[1, :NG, :] / jnp.maximum(pc_r[1, :NG, :], 1.0)
        cc = jnp.concatenate([mc, mo], axis=1)
        h1 = jax.nn.relu(_dot(cc, w_r[...]) + b_r[...])
        h2 = jax.nn.relu(_dot(h1, w_r[...]) + b_r[...])
        o_r[...] = _dot(h2, ow_r[...]) + ob_r[...]

    return pl.pallas_call(
        body,
        out_shape=jax.ShapeDtypeStruct((NG, 128), F32),
    )(psum, pcnt, lin_W, lin_b, out_Wp, out_bp)


def kernel(x_constraint, x_operator, edge_index_co, edge_index_oc,
           batch_constraint, batch_operator, params):
    p = params

    def padn(a):
        return jnp.pad(a, ((0, NP - N), (0, 0)))

    x_all = jnp.stack([padn(x_constraint), padn(x_operator)])

    def edges(e):
        src = jnp.pad(e[0].astype(jnp.int32), (0, EPAD - E))
        dst = jnp.pad(e[1].astype(jnp.int32), (0, EPAD - E),
                      constant_values=NP - 1)
        return (src.reshape(NSUB, NSTEP, 128), dst.reshape(NSUB, NSTEP, 128))

    s0, d0 = edges(edge_index_oc)
    s1, d1 = edges(edge_index_co)
    src_idx = jnp.stack([s0, s1])
    dst_idx = jnp.stack([d0, d1])

    def st(name):
        return jnp.stack([p[name % 'constraint'], p[name % 'operator']])

    lin_W = st('lin_W_%s')
    lin_b = st('lin_b_%s').reshape(2, 1, H)
    Wi = st('gru_Wi_%s')
    bi = st('gru_bi_%s').reshape(2, 1, 3 * H)
    bh = st('gru_bh_%s').reshape(2, 1, 3 * H)
    W_msg = [jnp.stack([p['W_msg_constraint_%d' % l], p['W_msg_operator_%d' % l]])
             for l in range(2)]
    W_self = [jnp.stack([p['W_self_constraint_%d' % l], p['W_self_operator_%d' % l]])
              for l in range(2)]
    b_conv = [jnp.stack([p['b_conv_constraint_%d' % l], p['b_conv_operator_%d' % l]]
                        ).reshape(2, 1, H) for l in range(2)]

    x1, msg0 = _tc_init(x_all, lin_W, lin_b, W_msg[0])

    cnt = _tc_count(dst_idx.reshape(2, EPAD, 1))
    cntr = cnt.reshape(2, 128 * 128)[:, :NP].reshape(2, NP, 1)

    acc0 = _seg_sum_sc(msg0, src_idx, dst_idx)
    x2, msg1 = _tc_conv_gru(x1, acc0, cntr, W_self[0], b_conv[0],
                            Wi, bi, bh, W_msg[1])

    acc1 = _seg_sum_sc(msg1, src_idx, dst_idx)
    x3, _ = _tc_conv_gru(x2, acc1, cntr, W_self[1], b_conv[1],
                         Wi, bi, bh, None)

    def padb(b):
        return jnp.pad(b.astype(jnp.int32), (0, NP - N), constant_values=NG)

    bids = jnp.stack([padb(batch_constraint), padb(batch_operator)]
                     ).reshape(2, NP, 1)
    psum, pcnt = _tc_pool(x3, bids)

    out_Wp = jnp.pad(p['out_W'], ((0, 0), (0, 128 - 2)))
    out_bp = jnp.pad(p['out_b'], (0, 128 - 2)).reshape(1, 128)
    out = _tc_mlp(psum, pcnt, p['lin_W'], p['lin_b'].reshape(1, 2 * H),
                  out_Wp, out_bp)
    return out[:, :2]


# speedup vs baseline: 2.5822x; 1.0401x over previous
import functools

import jax
import jax.numpy as jnp
from jax import lax
from jax.experimental import pallas as pl
from jax.experimental.pallas import tpu as pltpu
from jax.experimental.pallas import tpu_sc as plsc

N = 10000
NP = 10240
H = 128
E = 320000
NSUB = 16
NSTEP = 160
NCH = 16
EPAD = NSUB * NSTEP * 128
NG = 64
BLK = 1280
NBLK = NP // BLK
F32 = jnp.float32
_P = jax.lax.Precision.HIGHEST


def _dot(a, b):
    return jnp.dot(a, b, preferred_element_type=F32, precision=_P)


def _seg_sum_sc(msg_all, src_idx, dst_idx):
    mesh = plsc.VectorSubcoreMesh(core_axis_name="c", subcore_axis_name="s")

    @functools.partial(
        pl.kernel,
        out_type=jax.ShapeDtypeStruct((2, NP, H), F32),
        mesh=mesh,
        scratch_types=[
            pltpu.VMEM((NCH, 128), jnp.int32),
            pltpu.VMEM((NCH, 128), jnp.int32),
            pltpu.VMEM((128, H), F32),
            pltpu.VMEM((128, H), F32),
            pltpu.VMEM_SHARED((NP, H), F32),
            pltpu.SemaphoreType.DMA,
            pltpu.SemaphoreType.DMA,
        ],
    )
    def k(msg_hbm, src_hbm, dst_hbm, acc_out,
          src_v, dst_v, rows_v, rows1_v, acc_sp, sem, sem1):
        c = lax.axis_index("c")
        s = lax.axis_index("s")
        t_src = 1 - c
        z16 = jnp.zeros((16,), F32)

        def zero_rows(i, carry):
            rows_v[i // 8, pl.ds((i % 8) * 16, 16)] = z16
            return carry

        lax.fori_loop(0, 128 * 8, zero_rows, 0)

        for b in range(5):
            pltpu.sync_copy(rows_v, acc_sp.at[pl.ds(s * 640 + b * 128, 128)])
        plsc.subcore_barrier()

        bufs = (rows_v, rows1_v)
        sems = (sem, sem1)

        def chunk(jj, carry):
            pltpu.sync_copy(src_hbm.at[c, s, pl.ds(jj * NCH, NCH)], src_v)
            pltpu.sync_copy(dst_hbm.at[c, s, pl.ds(jj * NCH, NCH)], dst_v)
            descs = [None] * NCH
            descs[0] = pltpu.async_copy(
                msg_hbm.at[t_src].at[src_v.at[0]], bufs[0], sems[0])
            for m in range(NCH):
                if m + 1 < NCH:
                    descs[m + 1] = pltpu.async_copy(
                        msg_hbm.at[t_src].at[src_v.at[m + 1]],
                        bufs[(m + 1) % 2], sems[(m + 1) % 2])
                descs[m].wait()
                pltpu.sync_copy(bufs[m % 2], acc_sp.at[dst_v.at[m]], add=True)
            return carry

        lax.fori_loop(0, NSTEP // NCH, chunk, 0)
        plsc.subcore_barrier()

        for b in range(5):
            pltpu.sync_copy(acc_sp.at[pl.ds(s * 640 + b * 128, 128)],
                            acc_out.at[c, pl.ds(s * 640 + b * 128, 128)])

    return k(msg_all, src_idx, dst_idx)


def _tc_count(dst_flat):
    cblk = 2048

    def body(d_r, o_r):
        i = pl.program_id(1)
        d = d_r[0]
        g = lax.broadcasted_iota(jnp.int32, (1, 128), 1)
        ohr = (lax.shift_right_logical(d, 7) == g).astype(jnp.bfloat16)
        ohc = (lax.bitwise_and(d, 127) == g).astype(jnp.bfloat16)
        dn = (((0,), (0,)), ((), ()))
        cc = lax.dot_general(ohr, ohc, dn, preferred_element_type=F32)

        @pl.when(i == 0)
        def _():
            o_r[0] = cc

        @pl.when(i > 0)
        def _():
            o_r[0] += cc

    return pl.pallas_call(
        body,
        grid=(2, EPAD // cblk),
        in_specs=[pl.BlockSpec((1, cblk, 1), lambda t, i: (t, i, 0))],
        out_specs=pl.BlockSpec((1, 128, 128), lambda t, i: (t, 0, 0)),
        out_shape=jax.ShapeDtypeStruct((2, 128, 128), F32),
    )(dst_flat)


def _tc_init(x_all, lin_W, lin_b, W_msg0):
    def body(x_r, w_r, b_r, wm_r, x1_r, msg_r):
        y = _dot(x_r[0], w_r[0]) + b_r[0]
        x1_r[0] = y
        msg_r[0] = _dot(y, wm_r[0])

    return pl.pallas_call(
        body,
        grid=(2, NBLK),
        in_specs=[
            pl.BlockSpec((1, BLK, H), lambda t, i: (t, i, 0)),
            pl.BlockSpec((1, H, H), lambda t, i: (t, 0, 0)),
            pl.BlockSpec((1, 1, H), lambda t, i: (t, 0, 0)),
            pl.BlockSpec((1, H, H), lambda t, i: (t, 0, 0)),
        ],
        out_specs=[
            pl.BlockSpec((1, BLK, H), lambda t, i: (t, i, 0)),
            pl.BlockSpec((1, BLK, H), lambda t, i: (t, i, 0)),
        ],
        out_shape=[
            jax.ShapeDtypeStruct((2, NP, H), F32),
            jax.ShapeDtypeStruct((2, NP, H), F32),
        ],
    )(x_all, lin_W, lin_b, W_msg0)


def _tc_conv_gru(x_all, acc, cntr, W_self, b_conv, Wi, bi, bh, W_msg_next):
    with_msg = W_msg_next is not None

    def body(x_r, a_r, c_r, ws_r, bc_r, wi_r, bi_r, bh_r, *rest):
        if with_msg:
            wm_r, xo_r, mo_r = rest
        else:
            (xo_r,) = rest
        m = a_r[0] * (1.0 / jnp.maximum(c_r[0], 1.0))
        h = jax.nn.relu(_dot(x_r[0], ws_r[0]) + m + bc_r[0])
        gi = _dot(h, wi_r[0]) + bi_r[0]
        bhv = bh_r[0]
        r = jax.nn.sigmoid(gi[:, :H] + bhv[:, :H])
        z = jax.nn.sigmoid(gi[:, H:2 * H] + bhv[:, H:2 * H])
        n = jnp.tanh(gi[:, 2 * H:] + r * bhv[:, 2 * H:])
        xn = (1.0 - z) * n
        xo_r[0] = xn
        if with_msg:
            mo_r[0] = _dot(xn, wm_r[0])

    in_specs = [
        pl.BlockSpec((1, BLK, H), lambda t, i: (t, i, 0)),
        pl.BlockSpec((1, BLK, H), lambda t, i: (t, i, 0)),
        pl.BlockSpec((1, BLK, 1), lambda t, i: (t, i, 0)),
        pl.BlockSpec((1, H, H), lambda t, i: (t, 0, 0)),
        pl.BlockSpec((1, 1, H), lambda t, i: (t, 0, 0)),
        pl.BlockSpec((1, H, 3 * H), lambda t, i: (t, 0, 0)),
        pl.BlockSpec((1, 1, 3 * H), lambda t, i: (t, 0, 0)),
        pl.BlockSpec((1, 1, 3 * H), lambda t, i: (t, 0, 0)),
    ]
    out_specs = [pl.BlockSpec((1, BLK, H), lambda t, i: (t, i, 0))]
    out_shape = [jax.ShapeDtypeStruct((2, NP, H), F32)]
    args = [x_all, acc, cntr, W_self, b_conv, Wi, bi, bh]
    if with_msg:
        in_specs.append(pl.BlockSpec((1, H, H), lambda t, i: (t, 0, 0)))
        out_specs.append(pl.BlockSpec((1, BLK, H), lambda t, i: (t, i, 0)))
        out_shape.append(jax.ShapeDtypeStruct((2, NP, H), F32))
        args.append(W_msg_next)

    res = pl.pallas_call(
        body, grid=(2, NBLK), in_specs=in_specs,
        out_specs=out_specs, out_shape=out_shape,
    )(*args)
    return res if with_msg else (res[0], None)


def _tc_pool(x_all, bids):
    def body(x_r, b_r, ps_r, pc_r):
        i = pl.program_id(1)
        g = lax.broadcasted_iota(jnp.int32, (1, 128), 1)
        oh = (b_r[0] == g).astype(F32)
        dn = (((0,), (0,)), ((), ()))
        ps = lax.dot_general(oh, x_r[0], dn,
                             preferred_element_type=F32, precision=_P)
        pc = lax.dot_general(oh, jnp.ones((BLK, 128), F32), dn,
                             preferred_element_type=F32, precision=_P)

        @pl.when(i == 0)
        def _():
            ps_r[0] = ps
            pc_r[0] = pc

        @pl.when(i > 0)
        def _():
            ps_r[0] += ps
            pc_r[0] += pc

    return pl.pallas_call(
        body,
        grid=(2, NBLK),
        in_specs=[
            pl.BlockSpec((1, BLK, H), lambda t, i: (t, i, 0)),
            pl.BlockSpec((1, BLK, 1), lambda t, i: (t, i, 0)),
        ],
        out_specs=[
            pl.BlockSpec((1, 128, 128), lambda t, i: (t, 0, 0)),
            pl.BlockSpec((1, 128, 128), lambda t, i: (t, 0, 0)),
        ],
        out_shape=[
            jax.ShapeDtypeStruct((2, 128, 128), F32),
            jax.ShapeDtypeStruct((2, 128, 128), F32),
        ],
    )(x_all, bids)


def _tc_mlp(psum, pcnt, lin_W, lin_b, out_Wp, out_bp):
    def body(ps_r, pc_r, w_r, b_r, ow_r, ob_r, o_r):
        mc = ps_r[0, :NG, :] / jnp.maximum(pc_r[0, :NG, :], 1.0)
        mo = ps_r[1, :NG, :] / jnp.maximum(pc_r[1, :NG, :], 1.0)
        cc = jnp.concatenate([mc, mo], axis=1)
        h1 = jax.nn.relu(_dot(cc, w_r[...]) + b_r[...])
        h2 = jax.nn.relu(_dot(h1, w_r[...]) + b_r[...])
        o_r[...] = _dot(h2, ow_r[...]) + ob_r[...]

    return pl.pallas_call(
        body,
        out_shape=jax.ShapeDtypeStruct((NG, 128), F32),
    )(psum, pcnt, lin_W, lin_b, out_Wp, out_bp)


def kernel(x_constraint, x_operator, edge_index_co, edge_index_oc,
           batch_constraint, batch_operator, params):
    p = params

    def padn(a):
        return jnp.pad(a, ((0, NP - N), (0, 0)))

    x_all = jnp.stack([padn(x_constraint), padn(x_operator)])

    def edges(e):
        src = jnp.pad(e[0].astype(jnp.int32), (0, EPAD - E))
        dst = jnp.pad(e[1].astype(jnp.int32), (0, EPAD - E),
                      constant_values=NP - 1)
        return (src.reshape(NSUB, NSTEP, 128), dst.reshape(NSUB, NSTEP, 128))

    s0, d0 = edges(edge_index_oc)
    s1, d1 = edges(edge_index_co)
    src_idx = jnp.stack([s0, s1])
    dst_idx = jnp.stack([d0, d1])

    def st(name):
        return jnp.stack([p[name % 'constraint'], p[name % 'operator']])

    lin_W = st('lin_W_%s')
    lin_b = st('lin_b_%s').reshape(2, 1, H)
    Wi = st('gru_Wi_%s')
    bi = st('gru_bi_%s').reshape(2, 1, 3 * H)
    bh = st('gru_bh_%s').reshape(2, 1, 3 * H)
    W_msg = [jnp.stack([p['W_msg_constraint_%d' % l], p['W_msg_operator_%d' % l]])
             for l in range(2)]
    W_self = [jnp.stack([p['W_self_constraint_%d' % l], p['W_self_operator_%d' % l]])
              for l in range(2)]
    b_conv = [jnp.stack([p['b_conv_constraint_%d' % l], p['b_conv_operator_%d' % l]]
                        ).reshape(2, 1, H) for l in range(2)]

    x1, msg0 = _tc_init(x_all, lin_W, lin_b, W_msg[0])

    cnt = _tc_count(dst_idx.reshape(2, EPAD, 1))
    cntr = cnt.reshape(2, 128 * 128)[:, :NP].reshape(2, NP, 1)

    acc0 = _seg_sum_sc(msg0, src_idx, dst_idx)
    x2, msg1 = _tc_conv_gru(x1, acc0, cntr, W_self[0], b_conv[0],
                            Wi, bi, bh, W_msg[1])

    acc1 = _seg_sum_sc(msg1, src_idx, dst_idx)
    x3, _ = _tc_conv_gru(x2, acc1, cntr, W_self[1], b_conv[1],
                         Wi, bi, bh, None)

    def padb(b):
        return jnp.pad(b.astype(jnp.int32), (0, NP - N), constant_values=NG)

    bids = jnp.stack([padb(batch_constraint), padb(batch_operator)]
                     ).reshape(2, NP, 1)
    psum, pcnt = _tc_pool(x3, bids)

    out_Wp = jnp.pad(p['out_W'], ((0, 0), (0, 128 - 2)))
    out_bp = jnp.pad(p['out_b'], (0, 128 - 2)).reshape(1, 128)
    out = _tc_mlp(psum, pcnt, p['lin_W'], p['lin_b'].reshape(1, 2 * H),
                  out_Wp, out_bp)
    return out[:, :2]


# speedup vs baseline: 2.7317x; 1.0579x over previous
import functools

import jax
import jax.numpy as jnp
from jax import lax
from jax.experimental import pallas as pl
from jax.experimental.pallas import tpu as pltpu
from jax.experimental.pallas import tpu_sc as plsc

N = 10000
NP = 10240
H = 128
E = 320000
NSUB = 16
NSTEP = 160
NCH = 16
EPAD = NSUB * NSTEP * 128
NG = 64
BLK = 1280
NBLK = NP // BLK
F32 = jnp.float32
_P = jax.lax.Precision.DEFAULT


def _dot(a, b):
    return jnp.dot(a, b, preferred_element_type=F32, precision=_P)


def _seg_sum_sc(msg_all, src_idx, dst_idx):
    mesh = plsc.VectorSubcoreMesh(core_axis_name="c", subcore_axis_name="s")

    @functools.partial(
        pl.kernel,
        out_type=jax.ShapeDtypeStruct((2, NP, H), F32),
        mesh=mesh,
        scratch_types=[
            pltpu.VMEM((NCH, 128), jnp.int32),
            pltpu.VMEM((NCH, 128), jnp.int32),
            pltpu.VMEM((128, H), F32),
            pltpu.VMEM((128, H), F32),
            pltpu.VMEM_SHARED((NP, H), F32),
            pltpu.SemaphoreType.DMA,
            pltpu.SemaphoreType.DMA,
        ],
    )
    def k(msg_hbm, src_hbm, dst_hbm, acc_out,
          src_v, dst_v, rows_v, rows1_v, acc_sp, sem, sem1):
        c = lax.axis_index("c")
        s = lax.axis_index("s")
        t_src = 1 - c
        z16 = jnp.zeros((16,), F32)

        def zero_rows(i, carry):
            rows_v[i // 8, pl.ds((i % 8) * 16, 16)] = z16
            return carry

        lax.fori_loop(0, 128 * 8, zero_rows, 0)

        for b in range(5):
            pltpu.sync_copy(rows_v, acc_sp.at[pl.ds(s * 640 + b * 128, 128)])
        plsc.subcore_barrier()

        bufs = (rows_v, rows1_v)
        sems = (sem, sem1)

        def chunk(jj, carry):
            pltpu.sync_copy(src_hbm.at[c, s, pl.ds(jj * NCH, NCH)], src_v)
            pltpu.sync_copy(dst_hbm.at[c, s, pl.ds(jj * NCH, NCH)], dst_v)
            descs = [None] * NCH
            descs[0] = pltpu.async_copy(
                msg_hbm.at[t_src].at[src_v.at[0]], bufs[0], sems[0])
            for m in range(NCH):
                if m + 1 < NCH:
                    descs[m + 1] = pltpu.async_copy(
                        msg_hbm.at[t_src].at[src_v.at[m + 1]],
                        bufs[(m + 1) % 2], sems[(m + 1) % 2])
                descs[m].wait()
                pltpu.sync_copy(bufs[m % 2], acc_sp.at[dst_v.at[m]], add=True)
            return carry

        lax.fori_loop(0, NSTEP // NCH, chunk, 0)
        plsc.subcore_barrier()

        for b in range(5):
            pltpu.sync_copy(acc_sp.at[pl.ds(s * 640 + b * 128, 128)],
                            acc_out.at[c, pl.ds(s * 640 + b * 128, 128)])

    return k(msg_all, src_idx, dst_idx)


def _tc_count(dst_flat):
    cblk = 2048

    def body(d_r, o_r):
        i = pl.program_id(1)
        d = d_r[0]
        g = lax.broadcasted_iota(jnp.int32, (1, 128), 1)
        ohr = (lax.shift_right_logical(d, 7) == g).astype(jnp.bfloat16)
        ohc = (lax.bitwise_and(d, 127) == g).astype(jnp.bfloat16)
        dn = (((0,), (0,)), ((), ()))
        cc = lax.dot_general(ohr, ohc, dn, preferred_element_type=F32)

        @pl.when(i == 0)
        def _():
            o_r[0] = cc

        @pl.when(i > 0)
        def _():
            o_r[0] += cc

    return pl.pallas_call(
        body,
        grid=(2, EPAD // cblk),
        in_specs=[pl.BlockSpec((1, cblk, 1), lambda t, i: (t, i, 0))],
        out_specs=pl.BlockSpec((1, 128, 128), lambda t, i: (t, 0, 0)),
        out_shape=jax.ShapeDtypeStruct((2, 128, 128), F32),
    )(dst_flat)


def _tc_init(x_all, lin_W, lin_b, W_msg0):
    def body(x_r, w_r, b_r, wm_r, x1_r, msg_r):
        y = _dot(x_r[0], w_r[0]) + b_r[0]
        x1_r[0] = y
        msg_r[0] = _dot(y, wm_r[0])

    return pl.pallas_call(
        body,
        grid=(2, NBLK),
        in_specs=[
            pl.BlockSpec((1, BLK, H), lambda t, i: (t, i, 0)),
            pl.BlockSpec((1, H, H), lambda t, i: (t, 0, 0)),
            pl.BlockSpec((1, 1, H), lambda t, i: (t, 0, 0)),
            pl.BlockSpec((1, H, H), lambda t, i: (t, 0, 0)),
        ],
        out_specs=[
            pl.BlockSpec((1, BLK, H), lambda t, i: (t, i, 0)),
            pl.BlockSpec((1, BLK, H), lambda t, i: (t, i, 0)),
        ],
        out_shape=[
            jax.ShapeDtypeStruct((2, NP, H), F32),
            jax.ShapeDtypeStruct((2, NP, H), F32),
        ],
    )(x_all, lin_W, lin_b, W_msg0)


def _tc_conv_gru(x_all, acc, cntr, W_self, b_conv, Wi, bi, bh, W_msg_next):
    with_msg = W_msg_next is not None

    def body(x_r, a_r, c_r, ws_r, bc_r, wi_r, bi_r, bh_r, *rest):
        if with_msg:
            wm_r, xo_r, mo_r = rest
        else:
            (xo_r,) = rest
        m = a_r[0] * (1.0 / jnp.maximum(c_r[0], 1.0))
        h = jax.nn.relu(_dot(x_r[0], ws_r[0]) + m + bc_r[0])
        gi = _dot(h, wi_r[0]) + bi_r[0]
        bhv = bh_r[0]
        r = jax.nn.sigmoid(gi[:, :H] + bhv[:, :H])
        z = jax.nn.sigmoid(gi[:, H:2 * H] + bhv[:, H:2 * H])
        n = jnp.tanh(gi[:, 2 * H:] + r * bhv[:, 2 * H:])
        xn = (1.0 - z) * n
        xo_r[0] = xn
        if with_msg:
            mo_r[0] = _dot(xn, wm_r[0])

    in_specs = [
        pl.BlockSpec((1, BLK, H), lambda t, i: (t, i, 0)),
        pl.BlockSpec((1, BLK, H), lambda t, i: (t, i, 0)),
        pl.BlockSpec((1, BLK, 1), lambda t, i: (t, i, 0)),
        pl.BlockSpec((1, H, H), lambda t, i: (t, 0, 0)),
        pl.BlockSpec((1, 1, H), lambda t, i: (t, 0, 0)),
        pl.BlockSpec((1, H, 3 * H), lambda t, i: (t, 0, 0)),
        pl.BlockSpec((1, 1, 3 * H), lambda t, i: (t, 0, 0)),
        pl.BlockSpec((1, 1, 3 * H), lambda t, i: (t, 0, 0)),
    ]
    out_specs = [pl.BlockSpec((1, BLK, H), lambda t, i: (t, i, 0))]
    out_shape = [jax.ShapeDtypeStruct((2, NP, H), F32)]
    args = [x_all, acc, cntr, W_self, b_conv, Wi, bi, bh]
    if with_msg:
        in_specs.append(pl.BlockSpec((1, H, H), lambda t, i: (t, 0, 0)))
        out_specs.append(pl.BlockSpec((1, BLK, H), lambda t, i: (t, i, 0)))
        out_shape.append(jax.ShapeDtypeStruct((2, NP, H), F32))
        args.append(W_msg_next)

    res = pl.pallas_call(
        body, grid=(2, NBLK), in_specs=in_specs,
        out_specs=out_specs, out_shape=out_shape,
    )(*args)
    return res if with_msg else (res[0], None)


def _tc_pool(x_all, bids):
    def body(x_r, b_r, ps_r, pc_r):
        i = pl.program_id(1)
        g = lax.broadcasted_iota(jnp.int32, (1, 128), 1)
        oh = (b_r[0] == g).astype(F32)
        dn = (((0,), (0,)), ((), ()))
        ps = lax.dot_general(oh, x_r[0], dn,
                             preferred_element_type=F32, precision=_P)
        pc = lax.dot_general(oh, jnp.ones((BLK, 128), F32), dn,
                             preferred_element_type=F32, precision=_P)

        @pl.when(i == 0)
        def _():
            ps_r[0] = ps
            pc_r[0] = pc

        @pl.when(i > 0)
        def _():
            ps_r[0] += ps
            pc_r[0] += pc

    return pl.pallas_call(
        body,
        grid=(2, NBLK),
        in_specs=[
            pl.BlockSpec((1, BLK, H), lambda t, i: (t, i, 0)),
            pl.BlockSpec((1, BLK, 1), lambda t, i: (t, i, 0)),
        ],
        out_specs=[
            pl.BlockSpec((1, 128, 128), lambda t, i: (t, 0, 0)),
            pl.BlockSpec((1, 128, 128), lambda t, i: (t, 0, 0)),
        ],
        out_shape=[
            jax.ShapeDtypeStruct((2, 128, 128), F32),
            jax.ShapeDtypeStruct((2, 128, 128), F32),
        ],
    )(x_all, bids)


def _tc_mlp(psum, pcnt, lin_W, lin_b, out_Wp, out_bp):
    def body(ps_r, pc_r, w_r, b_r, ow_r, ob_r, o_r):
        mc = ps_r[0, :NG, :] / jnp.maximum(pc_r[0, :NG, :], 1.0)
        mo = ps_r[1, :NG, :] / jnp.maximum(pc_r[1, :NG, :], 1.0)
        cc = jnp.concatenate([mc, mo], axis=1)
        h1 = jax.nn.relu(_dot(cc, w_r[...]) + b_r[...])
        h2 = jax.nn.relu(_dot(h1, w_r[...]) + b_r[...])
        o_r[...] = _dot(h2, ow_r[...]) + ob_r[...]

    return pl.pallas_call(
        body,
        out_shape=jax.ShapeDtypeStruct((NG, 128), F32),
    )(psum, pcnt, lin_W, lin_b, out_Wp, out_bp)


def kernel(x_constraint, x_operator, edge_index_co, edge_index_oc,
           batch_constraint, batch_operator, params):
    p = params

    def padn(a):
        return jnp.pad(a, ((0, NP - N), (0, 0)))

    x_all = jnp.stack([padn(x_constraint), padn(x_operator)])

    def edges(e):
        src = jnp.pad(e[0].astype(jnp.int32), (0, EPAD - E))
        dst = jnp.pad(e[1].astype(jnp.int32), (0, EPAD - E),
                      constant_values=NP - 1)
        return (src.reshape(NSUB, NSTEP, 128), dst.reshape(NSUB, NSTEP, 128))

    s0, d0 = edges(edge_index_oc)
    s1, d1 = edges(edge_index_co)
    src_idx = jnp.stack([s0, s1])
    dst_idx = jnp.stack([d0, d1])

    def st(name):
        return jnp.stack([p[name % 'constraint'], p[name % 'operator']])

    lin_W = st('lin_W_%s')
    lin_b = st('lin_b_%s').reshape(2, 1, H)
    Wi = st('gru_Wi_%s')
    bi = st('gru_bi_%s').reshape(2, 1, 3 * H)
    bh = st('gru_bh_%s').reshape(2, 1, 3 * H)
    W_msg = [jnp.stack([p['W_msg_constraint_%d' % l], p['W_msg_operator_%d' % l]])
             for l in range(2)]
    W_self = [jnp.stack([p['W_self_constraint_%d' % l], p['W_self_operator_%d' % l]])
              for l in range(2)]
    b_conv = [jnp.stack([p['b_conv_constraint_%d' % l], p['b_conv_operator_%d' % l]]
                        ).reshape(2, 1, H) for l in range(2)]

    x1, msg0 = _tc_init(x_all, lin_W, lin_b, W_msg[0])

    cnt = _tc_count(dst_idx.reshape(2, EPAD, 1))
    cntr = cnt.reshape(2, 128 * 128)[:, :NP].reshape(2, NP, 1)

    acc0 = _seg_sum_sc(msg0, src_idx, dst_idx)
    x2, msg1 = _tc_conv_gru(x1, acc0, cntr, W_self[0], b_conv[0],
                            Wi, bi, bh, W_msg[1])

    acc1 = _seg_sum_sc(msg1, src_idx, dst_idx)
    x3, _ = _tc_conv_gru(x2, acc1, cntr, W_self[1], b_conv[1],
                         Wi, bi, bh, None)

    def padb(b):
        return jnp.pad(b.astype(jnp.int32), (0, NP - N), constant_values=NG)

    bids = jnp.stack([padb(batch_constraint), padb(batch_operator)]
                     ).reshape(2, NP, 1)
    psum, pcnt = _tc_pool(x3, bids)

    out_Wp = jnp.pad(p['out_W'], ((0, 0), (0, 128 - 2)))
    out_bp = jnp.pad(p['out_b'], (0, 128 - 2)).reshape(1, 128)
    out = _tc_mlp(psum, pcnt, p['lin_W'], p['lin_b'].reshape(1, 2 * H),
                  out_Wp, out_bp)
    return out[:, :2]


# speedup vs baseline: 2.8754x; 1.0526x over previous
import functools

import jax
import jax.numpy as jnp
from jax import lax
from jax.experimental import pallas as pl
from jax.experimental.pallas import tpu as pltpu
from jax.experimental.pallas import tpu_sc as plsc

N = 10000
NP = 10240
H = 128
E = 320000
NSUB = 16
NSTEP = 160
NCH = 16
EPAD = NSUB * NSTEP * 128
NG = 64
BLK = 1280
NBLK = NP // BLK
F32 = jnp.float32
_P = jax.lax.Precision.DEFAULT


def _dot(a, b):
    return jnp.dot(a, b, preferred_element_type=F32, precision=_P)


def _seg_sum_sc(msg_all, src_idx, dst_idx):
    mesh = plsc.VectorSubcoreMesh(core_axis_name="c", subcore_axis_name="s")

    @functools.partial(
        pl.kernel,
        out_type=jax.ShapeDtypeStruct((2, NP, H), F32),
        mesh=mesh,
        scratch_types=[
            pltpu.VMEM((NCH, 128), jnp.int32),
            pltpu.VMEM((NCH, 128), jnp.int32),
            pltpu.VMEM((128, H), F32),
            pltpu.VMEM((128, H), F32),
            pltpu.VMEM_SHARED((NP, H), F32),
            pltpu.SemaphoreType.DMA,
            pltpu.SemaphoreType.DMA,
        ],
    )
    def k(msg_hbm, src_hbm, dst_hbm, acc_out,
          src_v, dst_v, rows_v, rows1_v, acc_sp, sem, sem1):
        c = lax.axis_index("c")
        s = lax.axis_index("s")
        t_src = 1 - c
        z16 = jnp.zeros((16,), F32)

        def zero_rows(i, carry):
            rows_v[i // 8, pl.ds((i % 8) * 16, 16)] = z16
            return carry

        lax.fori_loop(0, 128 * 8, zero_rows, 0)

        for b in range(5):
            pltpu.sync_copy(rows_v, acc_sp.at[pl.ds(s * 640 + b * 128, 128)])
        plsc.subcore_barrier()

        bufs = (rows_v, rows1_v)
        sems = (sem, sem1)

        def chunk(jj, carry):
            pltpu.sync_copy(src_hbm.at[c, s, pl.ds(jj * NCH, NCH)], src_v)
            pltpu.sync_copy(dst_hbm.at[c, s, pl.ds(jj * NCH, NCH)], dst_v)
            descs = [None] * NCH
            descs[0] = pltpu.async_copy(
                msg_hbm.at[t_src].at[src_v.at[0]], bufs[0], sems[0])
            for m in range(NCH):
                if m + 1 < NCH:
                    descs[m + 1] = pltpu.async_copy(
                        msg_hbm.at[t_src].at[src_v.at[m + 1]],
                        bufs[(m + 1) % 2], sems[(m + 1) % 2])
                descs[m].wait()
                pltpu.sync_copy(bufs[m % 2], acc_sp.at[dst_v.at[m]], add=True)
            return carry

        lax.fori_loop(0, NSTEP // NCH, chunk, 0)
        plsc.subcore_barrier()

        for b in range(5):
            pltpu.sync_copy(acc_sp.at[pl.ds(s * 640 + b * 128, 128)],
                            acc_out.at[c, pl.ds(s * 640 + b * 128, 128)])

    return k(msg_all, src_idx, dst_idx)


def _tc_count(dst_flat):
    cblk = 16384

    def body(d_r, o_r):
        i = pl.program_id(1)
        d = d_r[0]
        g = lax.broadcasted_iota(jnp.int32, (1, 128), 1)
        ohr = (lax.shift_right_logical(d, 7) == g).astype(jnp.bfloat16)
        ohc = (lax.bitwise_and(d, 127) == g).astype(jnp.bfloat16)
        dn = (((0,), (0,)), ((), ()))
        cc = lax.dot_general(ohr, ohc, dn, preferred_element_type=F32)

        @pl.when(i == 0)
        def _():
            o_r[0] = cc

        @pl.when(i > 0)
        def _():
            o_r[0] += cc

    return pl.pallas_call(
        body,
        grid=(2, EPAD // cblk),
        in_specs=[pl.BlockSpec((1, cblk, 1), lambda t, i: (t, i, 0))],
        out_specs=pl.BlockSpec((1, 128, 128), lambda t, i: (t, 0, 0)),
        out_shape=jax.ShapeDtypeStruct((2, 128, 128), F32),
    )(dst_flat)


def _tc_init(x_all, lin_W, lin_b, W_msg0):
    def body(x_r, w_r, b_r, wm_r, x1_r, msg_r):
        y = _dot(x_r[0], w_r[0]) + b_r[0]
        x1_r[0] = y
        msg_r[0] = _dot(y, wm_r[0])

    return pl.pallas_call(
        body,
        grid=(2, NBLK),
        in_specs=[
            pl.BlockSpec((1, BLK, H), lambda t, i: (t, i, 0)),
            pl.BlockSpec((1, H, H), lambda t, i: (t, 0, 0)),
            pl.BlockSpec((1, 1, H), lambda t, i: (t, 0, 0)),
            pl.BlockSpec((1, H, H), lambda t, i: (t, 0, 0)),
        ],
        out_specs=[
            pl.BlockSpec((1, BLK, H), lambda t, i: (t, i, 0)),
            pl.BlockSpec((1, BLK, H), lambda t, i: (t, i, 0)),
        ],
        out_shape=[
            jax.ShapeDtypeStruct((2, NP, H), F32),
            jax.ShapeDtypeStruct((2, NP, H), F32),
        ],
    )(x_all, lin_W, lin_b, W_msg0)


def _tc_conv_gru(x_all, acc, cntr, W_self, b_conv, Wi, bi, bh, W_msg_next):
    with_msg = W_msg_next is not None

    def body(x_r, a_r, c_r, ws_r, bc_r, wi_r, bi_r, bh_r, *rest):
        if with_msg:
            wm_r, xo_r, mo_r = rest
        else:
            (xo_r,) = rest
        m = a_r[0] * (1.0 / jnp.maximum(c_r[0], 1.0))
        h = jax.nn.relu(_dot(x_r[0], ws_r[0]) + m + bc_r[0])
        gi = _dot(h, wi_r[0]) + bi_r[0]
        bhv = bh_r[0]
        r = jax.nn.sigmoid(gi[:, :H] + bhv[:, :H])
        z = jax.nn.sigmoid(gi[:, H:2 * H] + bhv[:, H:2 * H])
        n = jnp.tanh(gi[:, 2 * H:] + r * bhv[:, 2 * H:])
        xn = (1.0 - z) * n
        xo_r[0] = xn
        if with_msg:
            mo_r[0] = _dot(xn, wm_r[0])

    in_specs = [
        pl.BlockSpec((1, BLK, H), lambda t, i: (t, i, 0)),
        pl.BlockSpec((1, BLK, H), lambda t, i: (t, i, 0)),
        pl.BlockSpec((1, BLK, 1), lambda t, i: (t, i, 0)),
        pl.BlockSpec((1, H, H), lambda t, i: (t, 0, 0)),
        pl.BlockSpec((1, 1, H), lambda t, i: (t, 0, 0)),
        pl.BlockSpec((1, H, 3 * H), lambda t, i: (t, 0, 0)),
        pl.BlockSpec((1, 1, 3 * H), lambda t, i: (t, 0, 0)),
        pl.BlockSpec((1, 1, 3 * H), lambda t, i: (t, 0, 0)),
    ]
    out_specs = [pl.BlockSpec((1, BLK, H), lambda t, i: (t, i, 0))]
    out_shape = [jax.ShapeDtypeStruct((2, NP, H), F32)]
    args = [x_all, acc, cntr, W_self, b_conv, Wi, bi, bh]
    if with_msg:
        in_specs.append(pl.BlockSpec((1, H, H), lambda t, i: (t, 0, 0)))
        out_specs.append(pl.BlockSpec((1, BLK, H), lambda t, i: (t, i, 0)))
        out_shape.append(jax.ShapeDtypeStruct((2, NP, H), F32))
        args.append(W_msg_next)

    res = pl.pallas_call(
        body, grid=(2, NBLK), in_specs=in_specs,
        out_specs=out_specs, out_shape=out_shape,
    )(*args)
    return res if with_msg else (res[0], None)


def _tc_pool(x_all, bids):
    def body(x_r, b_r, ps_r, pc_r):
        i = pl.program_id(1)
        g = lax.broadcasted_iota(jnp.int32, (1, 128), 1)
        oh = (b_r[0] == g).astype(F32)
        dn = (((0,), (0,)), ((), ()))
        ps = lax.dot_general(oh, x_r[0], dn,
                             preferred_element_type=F32, precision=_P)
        pc = lax.dot_general(oh, jnp.ones((BLK, 128), F32), dn,
                             preferred_element_type=F32, precision=_P)

        @pl.when(i == 0)
        def _():
            ps_r[0] = ps
            pc_r[0] = pc

        @pl.when(i > 0)
        def _():
            ps_r[0] += ps
            pc_r[0] += pc

    return pl.pallas_call(
        body,
        grid=(2, NBLK),
        in_specs=[
            pl.BlockSpec((1, BLK, H), lambda t, i: (t, i, 0)),
            pl.BlockSpec((1, BLK, 1), lambda t, i: (t, i, 0)),
        ],
        out_specs=[
            pl.BlockSpec((1, 128, 128), lambda t, i: (t, 0, 0)),
            pl.BlockSpec((1, 128, 128), lambda t, i: (t, 0, 0)),
        ],
        out_shape=[
            jax.ShapeDtypeStruct((2, 128, 128), F32),
            jax.ShapeDtypeStruct((2, 128, 128), F32),
        ],
    )(x_all, bids)


def _tc_mlp(psum, pcnt, lin_W, lin_b, out_Wp, out_bp):
    def body(ps_r, pc_r, w_r, b_r, ow_r, ob_r, o_r):
        mc = ps_r[0, :NG, :] / jnp.maximum(pc_r[0, :NG, :], 1.0)
        mo = ps_r[1, :NG, :] / jnp.maximum(pc_r[1, :NG, :], 1.0)
        cc = jnp.concatenate([mc, mo], axis=1)
        h1 = jax.nn.relu(_dot(cc, w_r[...]) + b_r[...])
        h2 = jax.nn.relu(_dot(h1, w_r[...]) + b_r[...])
        o_r[...] = _dot(h2, ow_r[...]) + ob_r[...]

    return pl.pallas_call(
        body,
        out_shape=jax.ShapeDtypeStruct((NG, 128), F32),
    )(psum, pcnt, lin_W, lin_b, out_Wp, out_bp)


def kernel(x_constraint, x_operator, edge_index_co, edge_index_oc,
           batch_constraint, batch_operator, params):
    p = params

    def padn(a):
        return jnp.pad(a, ((0, NP - N), (0, 0)))

    x_all = jnp.stack([padn(x_constraint), padn(x_operator)])

    def edges(e):
        src = jnp.pad(e[0].astype(jnp.int32), (0, EPAD - E))
        dst = jnp.pad(e[1].astype(jnp.int32), (0, EPAD - E),
                      constant_values=NP - 1)
        return (src.reshape(NSUB, NSTEP, 128), dst.reshape(NSUB, NSTEP, 128))

    s0, d0 = edges(edge_index_oc)
    s1, d1 = edges(edge_index_co)
    src_idx = jnp.stack([s0, s1])
    dst_idx = jnp.stack([d0, d1])

    def st(name):
        return jnp.stack([p[name % 'constraint'], p[name % 'operator']])

    lin_W = st('lin_W_%s')
    lin_b = st('lin_b_%s').reshape(2, 1, H)
    Wi = st('gru_Wi_%s')
    bi = st('gru_bi_%s').reshape(2, 1, 3 * H)
    bh = st('gru_bh_%s').reshape(2, 1, 3 * H)
    W_msg = [jnp.stack([p['W_msg_constraint_%d' % l], p['W_msg_operator_%d' % l]])
             for l in range(2)]
    W_self = [jnp.stack([p['W_self_constraint_%d' % l], p['W_self_operator_%d' % l]])
              for l in range(2)]
    b_conv = [jnp.stack([p['b_conv_constraint_%d' % l], p['b_conv_operator_%d' % l]]
                        ).reshape(2, 1, H) for l in range(2)]

    x1, msg0 = _tc_init(x_all, lin_W, lin_b, W_msg[0])

    cnt = _tc_count(dst_idx.reshape(2, EPAD, 1))
    cntr = cnt.reshape(2, 128 * 128)[:, :NP].reshape(2, NP, 1)

    acc0 = _seg_sum_sc(msg0, src_idx, dst_idx)
    x2, msg1 = _tc_conv_gru(x1, acc0, cntr, W_self[0], b_conv[0],
                            Wi, bi, bh, W_msg[1])

    acc1 = _seg_sum_sc(msg1, src_idx, dst_idx)
    x3, _ = _tc_conv_gru(x2, acc1, cntr, W_self[1], b_conv[1],
                         Wi, bi, bh, None)

    def padb(b):
        return jnp.pad(b.astype(jnp.int32), (0, NP - N), constant_values=NG)

    bids = jnp.stack([padb(batch_constraint), padb(batch_operator)]
                     ).reshape(2, NP, 1)
    psum, pcnt = _tc_pool(x3, bids)

    out_Wp = jnp.pad(p['out_W'], ((0, 0), (0, 128 - 2)))
    out_bp = jnp.pad(p['out_b'], (0, 128 - 2)).reshape(1, 128)
    out = _tc_mlp(psum, pcnt, p['lin_W'], p['lin_b'].reshape(1, 2 * H),
                  out_Wp, out_bp)
    return out[:, :2]


# speedup vs baseline: 2.8787x; 1.0011x over previous
import functools

import jax
import jax.numpy as jnp
from jax import lax
from jax.experimental import pallas as pl
from jax.experimental.pallas import tpu as pltpu
from jax.experimental.pallas import tpu_sc as plsc

N = 10000
NP = 10240
H = 128
E = 320000
NSUB = 16
NSTEP = 160
NCH = 16
EPAD = NSUB * NSTEP * 128
NG = 64
BLK = 1280
NBLK = NP // BLK
F32 = jnp.float32
_P = jax.lax.Precision.DEFAULT


def _dot(a, b):
    return jnp.dot(a, b, preferred_element_type=F32, precision=_P)


def _seg_sum_sc(msg_all, src_idx, dst_idx):
    mesh = plsc.VectorSubcoreMesh(core_axis_name="c", subcore_axis_name="s")

    @functools.partial(
        pl.kernel,
        out_type=jax.ShapeDtypeStruct((2, NP, H), F32),
        mesh=mesh,
        scratch_types=[
            pltpu.VMEM((NCH, 128), jnp.int32),
            pltpu.VMEM((NCH, 128), jnp.int32),
            pltpu.VMEM((128, H), F32),
            pltpu.VMEM((128, H), F32),
            pltpu.VMEM_SHARED((NP, H), F32),
            pltpu.SemaphoreType.DMA,
            pltpu.SemaphoreType.DMA,
        ],
    )
    def k(msg_hbm, src_hbm, dst_hbm, acc_out,
          src_v, dst_v, rows_v, rows1_v, acc_sp, sem, sem1):
        c = lax.axis_index("c")
        s = lax.axis_index("s")
        t_src = 1 - c
        z16 = jnp.zeros((16,), F32)

        def zero_rows(i, carry):
            rows_v[i // 8, pl.ds((i % 8) * 16, 16)] = z16
            return carry

        lax.fori_loop(0, 128 * 8, zero_rows, 0)

        for b in range(5):
            pltpu.sync_copy(rows_v, acc_sp.at[pl.ds(s * 640 + b * 128, 128)])
        plsc.subcore_barrier()

        bufs = (rows_v, rows1_v)
        sems = (sem, sem1)

        def chunk(jj, carry):
            pltpu.sync_copy(src_hbm.at[c, s, pl.ds(jj * NCH, NCH)], src_v)
            pltpu.sync_copy(dst_hbm.at[c, s, pl.ds(jj * NCH, NCH)], dst_v)
            descs = [None] * NCH
            descs[0] = pltpu.async_copy(
                msg_hbm.at[t_src].at[src_v.at[0]], bufs[0], sems[0])
            for m in range(NCH):
                if m + 1 < NCH:
                    descs[m + 1] = pltpu.async_copy(
                        msg_hbm.at[t_src].at[src_v.at[m + 1]],
                        bufs[(m + 1) % 2], sems[(m + 1) % 2])
                descs[m].wait()
                pltpu.sync_copy(bufs[m % 2], acc_sp.at[dst_v.at[m]], add=True)
            return carry

        lax.fori_loop(0, NSTEP // NCH, chunk, 0)
        plsc.subcore_barrier()

        for b in range(5):
            pltpu.sync_copy(acc_sp.at[pl.ds(s * 640 + b * 128, 128)],
                            acc_out.at[c, pl.ds(s * 640 + b * 128, 128)])

    return k(msg_all, src_idx, dst_idx)


def _tc_count(dst_flat):
    cblk = 16384

    def body(d_r, o_r):
        i = pl.program_id(1)
        d = d_r[0]
        g = lax.broadcasted_iota(jnp.int32, (1, 128), 1)
        ohr = (lax.shift_right_logical(d, 7) == g).astype(jnp.bfloat16)
        ohc = (lax.bitwise_and(d, 127) == g).astype(jnp.bfloat16)
        dn = (((0,), (0,)), ((), ()))
        cc = lax.dot_general(ohr, ohc, dn, preferred_element_type=F32)

        @pl.when(i == 0)
        def _():
            o_r[0] = cc

        @pl.when(i > 0)
        def _():
            o_r[0] += cc

    return pl.pallas_call(
        body,
        grid=(2, EPAD // cblk),
        in_specs=[pl.BlockSpec((1, cblk, 1), lambda t, i: (t, i, 0))],
        out_specs=pl.BlockSpec((1, 128, 128), lambda t, i: (t, 0, 0)),
        out_shape=jax.ShapeDtypeStruct((2, 128, 128), F32),
    )(dst_flat)


def _tc_init(x_all, lin_W, lin_b, W_msg0):
    def body(x_r, w_r, b_r, wm_r, x1_r, msg_r):
        y = _dot(x_r[0], w_r[0]) + b_r[0]
        x1_r[0] = y
        msg_r[0] = _dot(y, wm_r[0])

    return pl.pallas_call(
        body,
        grid=(2, NBLK),
        in_specs=[
            pl.BlockSpec((1, BLK, H), lambda t, i: (t, i, 0)),
            pl.BlockSpec((1, H, H), lambda t, i: (t, 0, 0)),
            pl.BlockSpec((1, 1, H), lambda t, i: (t, 0, 0)),
            pl.BlockSpec((1, H, H), lambda t, i: (t, 0, 0)),
        ],
        out_specs=[
            pl.BlockSpec((1, BLK, H), lambda t, i: (t, i, 0)),
            pl.BlockSpec((1, BLK, H), lambda t, i: (t, i, 0)),
        ],
        out_shape=[
            jax.ShapeDtypeStruct((2, NP, H), F32),
            jax.ShapeDtypeStruct((2, NP, H), F32),
        ],
    )(x_all, lin_W, lin_b, W_msg0)


def _tc_conv_gru(x_all, acc, cntr, W_self, b_conv, Wi, bi, bh, W_msg_next):
    with_msg = W_msg_next is not None

    def body(x_r, a_r, c_r, ws_r, bc_r, wi_r, bi_r, bh_r, *rest):
        if with_msg:
            wm_r, xo_r, mo_r = rest
        else:
            (xo_r,) = rest
        m = a_r[0] * (1.0 / jnp.maximum(c_r[0], 1.0))
        h = jax.nn.relu(_dot(x_r[0], ws_r[0]) + m + bc_r[0])
        gi = _dot(h, wi_r[0]) + bi_r[0]
        bhv = bh_r[0]
        r = jax.nn.sigmoid(gi[:, :H] + bhv[:, :H])
        z = jax.nn.sigmoid(gi[:, H:2 * H] + bhv[:, H:2 * H])
        n = jnp.tanh(gi[:, 2 * H:] + r * bhv[:, 2 * H:])
        xn = (1.0 - z) * n
        xo_r[0] = xn
        if with_msg:
            mo_r[0] = _dot(xn, wm_r[0])

    in_specs = [
        pl.BlockSpec((1, BLK, H), lambda t, i: (t, i, 0)),
        pl.BlockSpec((1, BLK, H), lambda t, i: (t, i, 0)),
        pl.BlockSpec((1, BLK, 1), lambda t, i: (t, i, 0)),
        pl.BlockSpec((1, H, H), lambda t, i: (t, 0, 0)),
        pl.BlockSpec((1, 1, H), lambda t, i: (t, 0, 0)),
        pl.BlockSpec((1, H, 3 * H), lambda t, i: (t, 0, 0)),
        pl.BlockSpec((1, 1, 3 * H), lambda t, i: (t, 0, 0)),
        pl.BlockSpec((1, 1, 3 * H), lambda t, i: (t, 0, 0)),
    ]
    out_specs = [pl.BlockSpec((1, BLK, H), lambda t, i: (t, i, 0))]
    out_shape = [jax.ShapeDtypeStruct((2, NP, H), F32)]
    args = [x_all, acc, cntr, W_self, b_conv, Wi, bi, bh]
    if with_msg:
        in_specs.append(pl.BlockSpec((1, H, H), lambda t, i: (t, 0, 0)))
        out_specs.append(pl.BlockSpec((1, BLK, H), lambda t, i: (t, i, 0)))
        out_shape.append(jax.ShapeDtypeStruct((2, NP, H), F32))
        args.append(W_msg_next)

    res = pl.pallas_call(
        body, grid=(2, NBLK), in_specs=in_specs,
        out_specs=out_specs, out_shape=out_shape,
    )(*args)
    return res if with_msg else (res[0], None)


def _tc_pool(x_all, bids):
    def body(x_r, b_r, ps_r, pc_r):
        i = pl.program_id(1)
        g = lax.broadcasted_iota(jnp.int32, (1, 128), 1)
        oh = (b_r[0] == g).astype(F32)
        dn = (((0,), (0,)), ((), ()))
        ps = lax.dot_general(oh, x_r[0], dn,
                             preferred_element_type=F32, precision=_P)
        pc = lax.dot_general(oh, jnp.ones((BLK, 128), F32), dn,
                             preferred_element_type=F32, precision=_P)

        @pl.when(i == 0)
        def _():
            ps_r[0] = ps
            pc_r[0] = pc

        @pl.when(i > 0)
        def _():
            ps_r[0] += ps
            pc_r[0] += pc

    return pl.pallas_call(
        body,
        grid=(2, NBLK),
        in_specs=[
            pl.BlockSpec((1, BLK, H), lambda t, i: (t, i, 0)),
            pl.BlockSpec((1, BLK, 1), lambda t, i: (t, i, 0)),
        ],
        out_specs=[
            pl.BlockSpec((1, 128, 128), lambda t, i: (t, 0, 0)),
            pl.BlockSpec((1, 128, 128), lambda t, i: (t, 0, 0)),
        ],
        out_shape=[
            jax.ShapeDtypeStruct((2, 128, 128), F32),
            jax.ShapeDtypeStruct((2, 128, 128), F32),
        ],
    )(x_all, bids)


def _tc_mlp(psum, pcnt, lin_W, lin_b, out_Wp, out_bp):
    def body(ps_r, pc_r, w_r, b_r, ow_r, ob_r, o_r):
        mc = ps_r[0, :NG, :] / jnp.maximum(pc_r[0, :NG, :], 1.0)
        mo = ps_r[1, :NG, :] / jnp.maximum(pc_r[1, :NG, :], 1.0)
        cc = jnp.concatenate([mc, mo], axis=1)
        h1 = jax.nn.relu(_dot(cc, w_r[...]) + b_r[...])
        h2 = jax.nn.relu(_dot(h1, w_r[...]) + b_r[...])
        o_r[...] = _dot(h2, ow_r[...]) + ob_r[...]

    return pl.pallas_call(
        body,
        out_shape=jax.ShapeDtypeStruct((NG, 128), F32),
    )(psum, pcnt, lin_W, lin_b, out_Wp, out_bp)


def kernel(x_constraint, x_operator, edge_index_co, edge_index_oc,
           batch_constraint, batch_operator, params):
    p = params

    def padn(a):
        return jnp.pad(a, ((0, NP - N), (0, 0)))

    x_all = jnp.stack([padn(x_constraint), padn(x_operator)])

    def edges(e):
        src = jnp.pad(e[0].astype(jnp.int32), (0, EPAD - E))
        dst = jnp.pad(e[1].astype(jnp.int32), (0, EPAD - E),
                      constant_values=NP - 1)
        return (src.reshape(NSUB, NSTEP, 128), dst.reshape(NSUB, NSTEP, 128))

    s0, d0 = edges(edge_index_oc)
    s1, d1 = edges(edge_index_co)
    src_idx = jnp.stack([s0, s1])
    dst_idx = jnp.stack([d0, d1])

    def st(name):
        return jnp.stack([p[name % 'constraint'], p[name % 'operator']])

    lin_W = st('lin_W_%s')
    lin_b = st('lin_b_%s').reshape(2, 1, H)
    Wi = st('gru_Wi_%s')
    bi = st('gru_bi_%s').reshape(2, 1, 3 * H)
    bh = st('gru_bh_%s').reshape(2, 1, 3 * H)
    W_msg = [jnp.stack([p['W_msg_constraint_%d' % l], p['W_msg_operator_%d' % l]])
             for l in range(2)]
    W_self = [jnp.stack([p['W_self_constraint_%d' % l], p['W_self_operator_%d' % l]])
              for l in range(2)]
    b_conv = [jnp.stack([p['b_conv_constraint_%d' % l], p['b_conv_operator_%d' % l]]
                        ).reshape(2, 1, H) for l in range(2)]

    x1, msg0 = _tc_init(x_all, lin_W, lin_b, W_msg[0])

    acc0 = _seg_sum_sc(msg0, src_idx, dst_idx)
    cnt = _tc_count(dst_idx.reshape(2, EPAD, 1))
    cntr = cnt.reshape(2, 128 * 128)[:, :NP].reshape(2, NP, 1)
    x2, msg1 = _tc_conv_gru(x1, acc0, cntr, W_self[0], b_conv[0],
                            Wi, bi, bh, W_msg[1])

    acc1 = _seg_sum_sc(msg1, src_idx, dst_idx)
    x3, _ = _tc_conv_gru(x2, acc1, cntr, W_self[1], b_conv[1],
                         Wi, bi, bh, None)

    def padb(b):
        return jnp.pad(b.astype(jnp.int32), (0, NP - N), constant_values=NG)

    bids = jnp.stack([padb(batch_constraint), padb(batch_operator)]
                     ).reshape(2, NP, 1)
    psum, pcnt = _tc_pool(x3, bids)

    out_Wp = jnp.pad(p['out_W'], ((0, 0), (0, 128 - 2)))
    out_bp = jnp.pad(p['out_b'], (0, 128 - 2)).reshape(1, 128)
    out = _tc_mlp(psum, pcnt, p['lin_W'], p['lin_b'].reshape(1, 2 * H),
                  out_Wp, out_bp)
    return out[:, :2]


# speedup vs baseline: 2.9031x; 1.0085x over previous
import functools

import jax
import jax.numpy as jnp
from jax import lax
from jax.experimental import pallas as pl
from jax.experimental.pallas import tpu as pltpu
from jax.experimental.pallas import tpu_sc as plsc

N = 10000
NP = 10240
H = 128
E = 320000
NSUB = 16
NSTEP = 160
NCH = 16
EPAD = NSUB * NSTEP * 128
NG = 64
BLK = 1280
NBLK = NP // BLK
F32 = jnp.float32
_P = jax.lax.Precision.DEFAULT


def _dot(a, b):
    return jnp.dot(a, b, preferred_element_type=F32, precision=_P)


def _seg_sum_sc(msg_all, src_idx, dst_idx):
    mesh = plsc.VectorSubcoreMesh(core_axis_name="c", subcore_axis_name="s")

    @functools.partial(
        pl.kernel,
        out_type=jax.ShapeDtypeStruct((2, NP, H), F32),
        mesh=mesh,
        scratch_types=[
            pltpu.VMEM((NCH, 128), jnp.int32),
            pltpu.VMEM((NCH, 128), jnp.int32),
            pltpu.VMEM((128, H), F32),
            pltpu.VMEM((128, H), F32),
            pltpu.VMEM_SHARED((NP, H), F32),
            pltpu.SemaphoreType.DMA,
            pltpu.SemaphoreType.DMA,
        ],
    )
    def k(msg_hbm, src_hbm, dst_hbm, acc_out,
          src_v, dst_v, rows_v, rows1_v, acc_sp, sem, sem1):
        c = lax.axis_index("c")
        s = lax.axis_index("s")
        t_src = 1 - c
        z16 = jnp.zeros((16,), F32)

        def zero_rows(i, carry):
            rows_v[i // 8, pl.ds((i % 8) * 16, 16)] = z16
            return carry

        lax.fori_loop(0, 128 * 8, zero_rows, 0)

        for b in range(5):
            pltpu.sync_copy(rows_v, acc_sp.at[pl.ds(s * 640 + b * 128, 128)])
        plsc.subcore_barrier()

        bufs = (rows_v, rows1_v)
        sems = (sem, sem1)

        def chunk(jj, carry):
            pltpu.sync_copy(src_hbm.at[c, s, pl.ds(jj * NCH, NCH)], src_v)
            pltpu.sync_copy(dst_hbm.at[c, s, pl.ds(jj * NCH, NCH)], dst_v)
            descs = [None] * NCH
            descs[0] = pltpu.async_copy(
                msg_hbm.at[t_src].at[src_v.at[0]], bufs[0], sems[0])
            for m in range(NCH):
                if m + 1 < NCH:
                    descs[m + 1] = pltpu.async_copy(
                        msg_hbm.at[t_src].at[src_v.at[m + 1]],
                        bufs[(m + 1) % 2], sems[(m + 1) % 2])
                descs[m].wait()
                pltpu.sync_copy(bufs[m % 2], acc_sp.at[dst_v.at[m]], add=True)
            return carry

        lax.fori_loop(0, NSTEP // NCH, chunk, 0)
        plsc.subcore_barrier()

        for b in range(5):
            pltpu.sync_copy(acc_sp.at[pl.ds(s * 640 + b * 128, 128)],
                            acc_out.at[c, pl.ds(s * 640 + b * 128, 128)])

    return k(msg_all, src_idx, dst_idx)


def _tc_count(dr, dc):
    cblk = 16384
    BF = jnp.bfloat16

    def body(dr_r, dc_r, o_r):
        i = pl.program_id(1)
        ones_row = jnp.ones((1, 128), BF)
        g = lax.broadcasted_iota(jnp.int32, (1, 128), 1).astype(F32)
        dn_b = (((1,), (0,)), ((), ()))
        rb = lax.dot_general(dr_r[0], ones_row, dn_b,
                             preferred_element_type=F32)
        cb = lax.dot_general(dc_r[0], ones_row, dn_b,
                             preferred_element_type=F32)
        ohr = (rb == g).astype(BF)
        ohc = (cb == g).astype(BF)
        dn = (((0,), (0,)), ((), ()))
        cc = lax.dot_general(ohr, ohc, dn, preferred_element_type=F32)

        @pl.when(i == 0)
        def _():
            o_r[0] = cc

        @pl.when(i > 0)
        def _():
            o_r[0] += cc

    return pl.pallas_call(
        body,
        grid=(2, EPAD // cblk),
        in_specs=[
            pl.BlockSpec((1, cblk, 1), lambda t, i: (t, i, 0)),
            pl.BlockSpec((1, cblk, 1), lambda t, i: (t, i, 0)),
        ],
        out_specs=pl.BlockSpec((1, 128, 128), lambda t, i: (t, 0, 0)),
        out_shape=jax.ShapeDtypeStruct((2, 128, 128), F32),
    )(dr, dc)


def _tc_init(x_all, lin_W, lin_b, W_msg0):
    def body(x_r, w_r, b_r, wm_r, x1_r, msg_r):
        y = _dot(x_r[0], w_r[0]) + b_r[0]
        x1_r[0] = y
        msg_r[0] = _dot(y, wm_r[0])

    return pl.pallas_call(
        body,
        grid=(2, NBLK),
        in_specs=[
            pl.BlockSpec((1, BLK, H), lambda t, i: (t, i, 0)),
            pl.BlockSpec((1, H, H), lambda t, i: (t, 0, 0)),
            pl.BlockSpec((1, 1, H), lambda t, i: (t, 0, 0)),
            pl.BlockSpec((1, H, H), lambda t, i: (t, 0, 0)),
        ],
        out_specs=[
            pl.BlockSpec((1, BLK, H), lambda t, i: (t, i, 0)),
            pl.BlockSpec((1, BLK, H), lambda t, i: (t, i, 0)),
        ],
        out_shape=[
            jax.ShapeDtypeStruct((2, NP, H), F32),
            jax.ShapeDtypeStruct((2, NP, H), F32),
        ],
    )(x_all, lin_W, lin_b, W_msg0)


def _tc_conv_gru(x_all, acc, cntr, W_self, b_conv, Wi, bi, bh, W_msg_next):
    with_msg = W_msg_next is not None

    def body(x_r, a_r, c_r, ws_r, bc_r, wi_r, bi_r, bh_r, *rest):
        if with_msg:
            wm_r, xo_r, mo_r = rest
        else:
            (xo_r,) = rest
        m = a_r[0] * (1.0 / jnp.maximum(c_r[0], 1.0))
        h = jax.nn.relu(_dot(x_r[0], ws_r[0]) + m + bc_r[0])
        gi = _dot(h, wi_r[0]) + bi_r[0]
        bhv = bh_r[0]
        r = jax.nn.sigmoid(gi[:, :H] + bhv[:, :H])
        z = jax.nn.sigmoid(gi[:, H:2 * H] + bhv[:, H:2 * H])
        n = jnp.tanh(gi[:, 2 * H:] + r * bhv[:, 2 * H:])
        xn = (1.0 - z) * n
        xo_r[0] = xn
        if with_msg:
            mo_r[0] = _dot(xn, wm_r[0])

    in_specs = [
        pl.BlockSpec((1, BLK, H), lambda t, i: (t, i, 0)),
        pl.BlockSpec((1, BLK, H), lambda t, i: (t, i, 0)),
        pl.BlockSpec((1, BLK, 1), lambda t, i: (t, i, 0)),
        pl.BlockSpec((1, H, H), lambda t, i: (t, 0, 0)),
        pl.BlockSpec((1, 1, H), lambda t, i: (t, 0, 0)),
        pl.BlockSpec((1, H, 3 * H), lambda t, i: (t, 0, 0)),
        pl.BlockSpec((1, 1, 3 * H), lambda t, i: (t, 0, 0)),
        pl.BlockSpec((1, 1, 3 * H), lambda t, i: (t, 0, 0)),
    ]
    out_specs = [pl.BlockSpec((1, BLK, H), lambda t, i: (t, i, 0))]
    out_shape = [jax.ShapeDtypeStruct((2, NP, H), F32)]
    args = [x_all, acc, cntr, W_self, b_conv, Wi, bi, bh]
    if with_msg:
        in_specs.append(pl.BlockSpec((1, H, H), lambda t, i: (t, 0, 0)))
        out_specs.append(pl.BlockSpec((1, BLK, H), lambda t, i: (t, i, 0)))
        out_shape.append(jax.ShapeDtypeStruct((2, NP, H), F32))
        args.append(W_msg_next)

    res = pl.pallas_call(
        body, grid=(2, NBLK), in_specs=in_specs,
        out_specs=out_specs, out_shape=out_shape,
    )(*args)
    return res if with_msg else (res[0], None)


def _tc_pool(x_all, bids):
    def body(x_r, b_r, ps_r, pc_r):
        i = pl.program_id(1)
        g = lax.broadcasted_iota(jnp.int32, (1, 128), 1)
        oh = (b_r[0] == g).astype(F32)
        dn = (((0,), (0,)), ((), ()))
        ps = lax.dot_general(oh, x_r[0], dn,
                             preferred_element_type=F32, precision=_P)
        pc = lax.dot_general(oh, jnp.ones((BLK, 128), F32), dn,
                             preferred_element_type=F32, precision=_P)

        @pl.when(i == 0)
        def _():
            ps_r[0] = ps
            pc_r[0] = pc

        @pl.when(i > 0)
        def _():
            ps_r[0] += ps
            pc_r[0] += pc

    return pl.pallas_call(
        body,
        grid=(2, NBLK),
        in_specs=[
            pl.BlockSpec((1, BLK, H), lambda t, i: (t, i, 0)),
            pl.BlockSpec((1, BLK, 1), lambda t, i: (t, i, 0)),
        ],
        out_specs=[
            pl.BlockSpec((1, 128, 128), lambda t, i: (t, 0, 0)),
            pl.BlockSpec((1, 128, 128), lambda t, i: (t, 0, 0)),
        ],
        out_shape=[
            jax.ShapeDtypeStruct((2, 128, 128), F32),
            jax.ShapeDtypeStruct((2, 128, 128), F32),
        ],
    )(x_all, bids)


def _tc_mlp(psum, pcnt, lin_W, lin_b, out_Wp, out_bp):
    def body(ps_r, pc_r, w_r, b_r, ow_r, ob_r, o_r):
        mc = ps_r[0, :NG, :] / jnp.maximum(pc_r[0, :NG, :], 1.0)
        mo = ps_r[1, :NG, :] / jnp.maximum(pc_r[1, :NG, :], 1.0)
        cc = jnp.concatenate([mc, mo], axis=1)
        h1 = jax.nn.relu(_dot(cc, w_r[...]) + b_r[...])
        h2 = jax.nn.relu(_dot(h1, w_r[...]) + b_r[...])
        o_r[...] = _dot(h2, ow_r[...]) + ob_r[...]

    return pl.pallas_call(
        body,
        out_shape=jax.ShapeDtypeStruct((NG, 128), F32),
    )(psum, pcnt, lin_W, lin_b, out_Wp, out_bp)


def kernel(x_constraint, x_operator, edge_index_co, edge_index_oc,
           batch_constraint, batch_operator, params):
    p = params

    def padn(a):
        return jnp.pad(a, ((0, NP - N), (0, 0)))

    x_all = jnp.stack([padn(x_constraint), padn(x_operator)])

    def edges(e):
        src = jnp.pad(e[0].astype(jnp.int32), (0, EPAD - E))
        dst = jnp.pad(e[1].astype(jnp.int32), (0, EPAD - E),
                      constant_values=NP - 1)
        return (src.reshape(NSUB, NSTEP, 128), dst.reshape(NSUB, NSTEP, 128))

    s0, d0 = edges(edge_index_oc)
    s1, d1 = edges(edge_index_co)
    src_idx = jnp.stack([s0, s1])
    dst_idx = jnp.stack([d0, d1])

    def st(name):
        return jnp.stack([p[name % 'constraint'], p[name % 'operator']])

    lin_W = st('lin_W_%s')
    lin_b = st('lin_b_%s').reshape(2, 1, H)
    Wi = st('gru_Wi_%s')
    bi = st('gru_bi_%s').reshape(2, 1, 3 * H)
    bh = st('gru_bh_%s').reshape(2, 1, 3 * H)
    W_msg = [jnp.stack([p['W_msg_constraint_%d' % l], p['W_msg_operator_%d' % l]])
             for l in range(2)]
    W_self = [jnp.stack([p['W_self_constraint_%d' % l], p['W_self_operator_%d' % l]])
              for l in range(2)]
    b_conv = [jnp.stack([p['b_conv_constraint_%d' % l], p['b_conv_operator_%d' % l]]
                        ).reshape(2, 1, H) for l in range(2)]

    x1, msg0 = _tc_init(x_all, lin_W, lin_b, W_msg[0])

    acc0 = _seg_sum_sc(msg0, src_idx, dst_idx)
    dflat = dst_idx.reshape(2, EPAD, 1)
    cnt = _tc_count((dflat >> 7).astype(jnp.bfloat16),
                    (dflat & 127).astype(jnp.bfloat16))
    cntr = cnt.reshape(2, 128 * 128)[:, :NP].reshape(2, NP, 1)
    x2, msg1 = _tc_conv_gru(x1, acc0, cntr, W_self[0], b_conv[0],
                            Wi, bi, bh, W_msg[1])

    acc1 = _seg_sum_sc(msg1, src_idx, dst_idx)
    x3, _ = _tc_conv_gru(x2, acc1, cntr, W_self[1], b_conv[1],
                         Wi, bi, bh, None)

    def padb(b):
        return jnp.pad(b.astype(jnp.int32), (0, NP - N), constant_values=NG)

    bids = jnp.stack([padb(batch_constraint), padb(batch_operator)]
                     ).reshape(2, NP, 1)
    psum, pcnt = _tc_pool(x3, bids)

    out_Wp = jnp.pad(p['out_W'], ((0, 0), (0, 128 - 2)))
    out_bp = jnp.pad(p['out_b'], (0, 128 - 2)).reshape(1, 128)
    out = _tc_mlp(psum, pcnt, p['lin_W'], p['lin_b'].reshape(1, 2 * H),
                  out_Wp, out_bp)
    return out[:, :2]


# speedup vs baseline: 2.9282x; 1.0087x over previous
import functools

import jax
import jax.numpy as jnp
from jax import lax
from jax.experimental import pallas as pl
from jax.experimental.pallas import tpu as pltpu
from jax.experimental.pallas import tpu_sc as plsc

N = 10000
NP = 10240
H = 128
E = 320000
NSUB = 16
NSTEP = 160
NCH = 16
EPAD = NSUB * NSTEP * 128
NG = 64
BLK = 1280
NBLK = NP // BLK
F32 = jnp.float32
_P = jax.lax.Precision.DEFAULT


def _dot(a, b):
    return jnp.dot(a, b, preferred_element_type=F32, precision=_P)


def _seg_sum_sc(msg_all, src_idx, dst_idx):
    mesh = plsc.VectorSubcoreMesh(core_axis_name="c", subcore_axis_name="s")

    @functools.partial(
        pl.kernel,
        out_type=jax.ShapeDtypeStruct((2, NP, H), F32),
        mesh=mesh,
        scratch_types=[
            pltpu.VMEM((NCH, 128), jnp.int32),
            pltpu.VMEM((NCH, 128), jnp.int32),
            pltpu.VMEM((128, H), F32),
            pltpu.VMEM((128, H), F32),
            pltpu.VMEM_SHARED((NP, H), F32),
            pltpu.SemaphoreType.DMA,
            pltpu.SemaphoreType.DMA,
        ],
    )
    def k(msg_hbm, src_hbm, dst_hbm, acc_out,
          src_v, dst_v, rows_v, rows1_v, acc_sp, sem, sem1):
        c = lax.axis_index("c")
        s = lax.axis_index("s")
        t_src = 1 - c
        z16 = jnp.zeros((16,), F32)

        def zero_rows(i, carry):
            rows_v[i // 8, pl.ds((i % 8) * 16, 16)] = z16
            return carry

        lax.fori_loop(0, 128 * 8, zero_rows, 0)

        for b in range(5):
            pltpu.sync_copy(rows_v, acc_sp.at[pl.ds(s * 640 + b * 128, 128)])
        plsc.subcore_barrier()

        bufs = (rows_v, rows1_v)
        sems = (sem, sem1)

        def chunk(jj, carry):
            pltpu.sync_copy(src_hbm.at[c, s, pl.ds(jj * NCH, NCH)], src_v)
            pltpu.sync_copy(dst_hbm.at[c, s, pl.ds(jj * NCH, NCH)], dst_v)
            descs = [None] * NCH
            descs[0] = pltpu.async_copy(
                msg_hbm.at[t_src].at[src_v.at[0]], bufs[0], sems[0])
            for m in range(NCH):
                if m + 1 < NCH:
                    descs[m + 1] = pltpu.async_copy(
                        msg_hbm.at[t_src].at[src_v.at[m + 1]],
                        bufs[(m + 1) % 2], sems[(m + 1) % 2])
                descs[m].wait()
                pltpu.sync_copy(bufs[m % 2], acc_sp.at[dst_v.at[m]], add=True)
            return carry

        lax.fori_loop(0, NSTEP // NCH, chunk, 0)
        plsc.subcore_barrier()

        for b in range(5):
            pltpu.sync_copy(acc_sp.at[pl.ds(s * 640 + b * 128, 128)],
                            acc_out.at[c, pl.ds(s * 640 + b * 128, 128)])

    return k(msg_all, src_idx, dst_idx)


def _tc_count(dr, dc):
    cblk = 16384
    BF = jnp.bfloat16

    def body(dr_r, dc_r, o_r):
        i = pl.program_id(1)
        ones_row = jnp.ones((1, 128), BF)
        g = lax.broadcasted_iota(jnp.int32, (1, 128), 1).astype(F32)
        dn_b = (((1,), (0,)), ((), ()))
        rb = lax.dot_general(dr_r[0], ones_row, dn_b,
                             preferred_element_type=F32)
        cb = lax.dot_general(dc_r[0], ones_row, dn_b,
                             preferred_element_type=F32)
        ohr = (rb == g).astype(BF)
        ohc = (cb == g).astype(BF)
        dn = (((0,), (0,)), ((), ()))
        cc = lax.dot_general(ohr, ohc, dn, preferred_element_type=F32)

        @pl.when(i == 0)
        def _():
            o_r[0] = cc

        @pl.when(i > 0)
        def _():
            o_r[0] += cc

    return pl.pallas_call(
        body,
        grid=(2, EPAD // cblk),
        in_specs=[
            pl.BlockSpec((1, cblk, 1), lambda t, i: (t, i, 0)),
            pl.BlockSpec((1, cblk, 1), lambda t, i: (t, i, 0)),
        ],
        out_specs=pl.BlockSpec((1, 128, 128), lambda t, i: (t, 0, 0)),
        out_shape=jax.ShapeDtypeStruct((2, 128, 128), F32),
    )(dr, dc)


def _tc_init(x_all, lin_W, lin_b, W_msg0):
    def body(x_r, w_r, b_r, wm_r, x1_r, msg_r):
        y = _dot(x_r[0], w_r[0]) + b_r[0]
        x1_r[0] = y
        msg_r[0] = _dot(y, wm_r[0])

    return pl.pallas_call(
        body,
        grid=(2, NBLK),
        in_specs=[
            pl.BlockSpec((1, BLK, H), lambda t, i: (t, i, 0)),
            pl.BlockSpec((1, H, H), lambda t, i: (t, 0, 0)),
            pl.BlockSpec((1, 1, H), lambda t, i: (t, 0, 0)),
            pl.BlockSpec((1, H, H), lambda t, i: (t, 0, 0)),
        ],
        out_specs=[
            pl.BlockSpec((1, BLK, H), lambda t, i: (t, i, 0)),
            pl.BlockSpec((1, BLK, H), lambda t, i: (t, i, 0)),
        ],
        out_shape=[
            jax.ShapeDtypeStruct((2, NP, H), F32),
            jax.ShapeDtypeStruct((2, NP, H), F32),
        ],
    )(x_all, lin_W, lin_b, W_msg0)


def _tc_conv_gru(x_all, acc, cntr, W_self, b_conv, Wi, bi, bh, W_msg_next,
                 bids=None):
    with_msg = W_msg_next is not None

    def body(x_r, a_r, c_r, ws_r, bc_r, wi_r, bi_r, bh_r, *rest):
        if with_msg:
            wm_r, xo_r, mo_r = rest
        else:
            b_r, ps_r, pc_r = rest
        m = a_r[0] * (1.0 / jnp.maximum(c_r[0], 1.0))
        h = jax.nn.relu(_dot(x_r[0], ws_r[0]) + m + bc_r[0])
        gi = _dot(h, wi_r[0]) + bi_r[0]
        bhv = bh_r[0]
        r = jax.nn.sigmoid(gi[:, :H] + bhv[:, :H])
        z = jax.nn.sigmoid(gi[:, H:2 * H] + bhv[:, H:2 * H])
        n = jnp.tanh(gi[:, 2 * H:] + r * bhv[:, 2 * H:])
        xn = (1.0 - z) * n
        if with_msg:
            xo_r[0] = xn
            mo_r[0] = _dot(xn, wm_r[0])
        else:
            i = pl.program_id(1)
            g = lax.broadcasted_iota(jnp.int32, (1, 128), 1)
            oh = (b_r[0] == g).astype(F32)
            dn = (((0,), (0,)), ((), ()))
            ps = lax.dot_general(oh, xn, dn, preferred_element_type=F32)
            pc = lax.dot_general(oh, jnp.ones((BLK, 128), F32), dn,
                                 preferred_element_type=F32)

            @pl.when(i == 0)
            def _():
                ps_r[0] = ps
                pc_r[0] = pc

            @pl.when(i > 0)
            def _():
                ps_r[0] += ps
                pc_r[0] += pc

    in_specs = [
        pl.BlockSpec((1, BLK, H), lambda t, i: (t, i, 0)),
        pl.BlockSpec((1, BLK, H), lambda t, i: (t, i, 0)),
        pl.BlockSpec((1, BLK, 1), lambda t, i: (t, i, 0)),
        pl.BlockSpec((1, H, H), lambda t, i: (t, 0, 0)),
        pl.BlockSpec((1, 1, H), lambda t, i: (t, 0, 0)),
        pl.BlockSpec((1, H, 3 * H), lambda t, i: (t, 0, 0)),
        pl.BlockSpec((1, 1, 3 * H), lambda t, i: (t, 0, 0)),
        pl.BlockSpec((1, 1, 3 * H), lambda t, i: (t, 0, 0)),
    ]
    args = [x_all, acc, cntr, W_self, b_conv, Wi, bi, bh]
    if with_msg:
        in_specs.append(pl.BlockSpec((1, H, H), lambda t, i: (t, 0, 0)))
        out_specs = [pl.BlockSpec((1, BLK, H), lambda t, i: (t, i, 0)),
                     pl.BlockSpec((1, BLK, H), lambda t, i: (t, i, 0))]
        out_shape = [jax.ShapeDtypeStruct((2, NP, H), F32),
                     jax.ShapeDtypeStruct((2, NP, H), F32)]
        args.append(W_msg_next)
    else:
        in_specs.append(pl.BlockSpec((1, BLK, 1), lambda t, i: (t, i, 0)))
        out_specs = [pl.BlockSpec((1, 128, 128), lambda t, i: (t, 0, 0)),
                     pl.BlockSpec((1, 128, 128), lambda t, i: (t, 0, 0))]
        out_shape = [jax.ShapeDtypeStruct((2, 128, 128), F32),
                     jax.ShapeDtypeStruct((2, 128, 128), F32)]
        args.append(bids)

    res = pl.pallas_call(
        body, grid=(2, NBLK), in_specs=in_specs,
        out_specs=out_specs, out_shape=out_shape,
    )(*args)
    return res


def _tc_mlp(psum, pcnt, lin_W, lin_b, out_Wp, out_bp):
    def body(ps_r, pc_r, w_r, b_r, ow_r, ob_r, o_r):
        mc = ps_r[0, :NG, :] / jnp.maximum(pc_r[0, :NG, :], 1.0)
        mo = ps_r[1, :NG, :] / jnp.maximum(pc_r[1, :NG, :], 1.0)
        cc = jnp.concatenate([mc, mo], axis=1)
        h1 = jax.nn.relu(_dot(cc, w_r[...]) + b_r[...])
        h2 = jax.nn.relu(_dot(h1, w_r[...]) + b_r[...])
        o_r[...] = _dot(h2, ow_r[...]) + ob_r[...]

    return pl.pallas_call(
        body,
        out_shape=jax.ShapeDtypeStruct((NG, 128), F32),
    )(psum, pcnt, lin_W, lin_b, out_Wp, out_bp)


def kernel(x_constraint, x_operator, edge_index_co, edge_index_oc,
           batch_constraint, batch_operator, params):
    p = params

    def padn(a):
        return jnp.pad(a, ((0, NP - N), (0, 0)))

    x_all = jnp.stack([padn(x_constraint), padn(x_operator)])

    def edges(e):
        src = jnp.pad(e[0].astype(jnp.int32), (0, EPAD - E))
        dst = jnp.pad(e[1].astype(jnp.int32), (0, EPAD - E),
                      constant_values=NP - 1)
        return (src.reshape(NSUB, NSTEP, 128), dst.reshape(NSUB, NSTEP, 128))

    s0, d0 = edges(edge_index_oc)
    s1, d1 = edges(edge_index_co)
    src_idx = jnp.stack([s0, s1])
    dst_idx = jnp.stack([d0, d1])

    def st(name):
        return jnp.stack([p[name % 'constraint'], p[name % 'operator']])

    lin_W = st('lin_W_%s')
    lin_b = st('lin_b_%s').reshape(2, 1, H)
    Wi = st('gru_Wi_%s')
    bi = st('gru_bi_%s').reshape(2, 1, 3 * H)
    bh = st('gru_bh_%s').reshape(2, 1, 3 * H)
    W_msg = [jnp.stack([p['W_msg_constraint_%d' % l], p['W_msg_operator_%d' % l]])
             for l in range(2)]
    W_self = [jnp.stack([p['W_self_constraint_%d' % l], p['W_self_operator_%d' % l]])
              for l in range(2)]
    b_conv = [jnp.stack([p['b_conv_constraint_%d' % l], p['b_conv_operator_%d' % l]]
                        ).reshape(2, 1, H) for l in range(2)]

    x1, msg0 = _tc_init(x_all, lin_W, lin_b, W_msg[0])

    acc0 = _seg_sum_sc(msg0, src_idx, dst_idx)
    dflat = dst_idx.reshape(2, EPAD, 1)
    cnt = _tc_count((dflat >> 7).astype(jnp.bfloat16),
                    (dflat & 127).astype(jnp.bfloat16))
    cntr = cnt.reshape(2, 128 * 128)[:, :NP].reshape(2, NP, 1)
    x2, msg1 = _tc_conv_gru(x1, acc0, cntr, W_self[0], b_conv[0],
                            Wi, bi, bh, W_msg[1])

    acc1 = _seg_sum_sc(msg1, src_idx, dst_idx)

    def padb(b):
        return jnp.pad(b.astype(jnp.int32), (0, NP - N), constant_values=NG)

    bids = jnp.stack([padb(batch_constraint), padb(batch_operator)]
                     ).reshape(2, NP, 1)
    psum, pcnt = _tc_conv_gru(x2, acc1, cntr, W_self[1], b_conv[1],
                              Wi, bi, bh, None, bids)

    out_Wp = jnp.pad(p['out_W'], ((0, 0), (0, 128 - 2)))
    out_bp = jnp.pad(p['out_b'], (0, 128 - 2)).reshape(1, 128)
    out = _tc_mlp(psum, pcnt, p['lin_W'], p['lin_b'].reshape(1, 2 * H),
                  out_Wp, out_bp)
    return out[:, :2]
